# baseline (speedup 1.0000x reference)
import functools

import jax
import jax.numpy as jnp
from jax import lax
from jax.experimental import pallas as pl
from jax.experimental.pallas import tpu as pltpu
from jax.experimental.pallas import tpu_sc as plsc

_N = 2048
_E = 32768
_D = 1024
_SCALE = 1.0 / float(_D) ** 0.5


_EPT = _E // 16
_NGRP = _EPT // 16
_OWN = 65536


def _sc_counts(s1, d1, s2, d2):
    mesh = plsc.VectorSubcoreMesh(core_axis_name="c", subcore_axis_name="s")
    out_t = jax.ShapeDtypeStruct((2 * _N * _N,), jnp.float32)
    scratch = [
        pltpu.VMEM((_EPT,), jnp.int32),
        pltpu.VMEM((_EPT,), jnp.int32),
        pltpu.VMEM((_EPT,), jnp.int32),
        pltpu.VMEM((_EPT,), jnp.float32),
        pltpu.VMEM((_E // 2,), jnp.int32),
        pltpu.VMEM((_E // 2,), jnp.float32),
        pltpu.VMEM((_OWN,), jnp.float32),
        pltpu.VMEM((32,), jnp.int32),
        pltpu.VMEM_SHARED((_E,), jnp.int32),
        pltpu.VMEM_SHARED((_E,), jnp.float32),
    ]
    @functools.partial(pl.kernel, out_type=out_t, mesh=mesh,
                       scratch_types=scratch,
                       compiler_params=pltpu.CompilerParams(
                           needs_layout_passes=False))
    def k(s1_hbm, d1_hbm, s2_hbm, d2_hbm, o_hbm,
          src_v, dst_v, fs_v, rv_v, cf_v, cv_v, own_v, rot_v, spm_f, spm_v):
        c = lax.axis_index("c")
        t = lax.axis_index("s")
        w = c * 16 + t
        base = t * _EPT
        iota16 = lax.iota(jnp.int32, 16)

        for l in range(2):
            sh = (s1_hbm, s2_hbm)[l]
            dh = (d1_hbm, d2_hbm)[l]
            pltpu.sync_copy(sh.at[pl.ds(base, _EPT)], src_v)
            pltpu.sync_copy(dh.at[pl.ds(base, _EPT)], dst_v)

            def pre(g, carry):
                sv = src_v[pl.ds(g * 16, 16)]
                dv = dst_v[pl.ds(g * 16, 16)]
                f = dv * _N + sv
                rot_v[pl.ds(0, 16)] = f
                rot_v[pl.ds(16, 16)] = f
                eqs = [f == rot_v[pl.ds(kk, 16)] for kk in range(1, 16)]
                ints = [jnp.where(e, 1, 0) for e in eqs]
                dups = [eqs[kk - 1] & (iota16 >= 16 - kk)
                        for kk in range(1, 16)]
                while len(ints) > 1:
                    ints = [a + b for a, b in zip(ints[::2], ints[1::2])
                            ] + ints[len(ints) & ~1:]
                    dups = [a | b for a, b in zip(dups[::2], dups[1::2])
                            ] + dups[len(dups) & ~1:]
                cnt = 1 + ints[0]
                dup = dups[0]
                fs_v[pl.ds(g * 16, 16)] = jnp.where(dup, _N * _N, f)
                rv_v[pl.ds(g * 16, 16)] = cnt.astype(jnp.float32)
                return carry
            lax.fori_loop(0, _NGRP, pre, 0)

            plsc.subcore_barrier()
            pltpu.sync_copy(fs_v, spm_f.at[pl.ds(base, _EPT)])
            pltpu.sync_copy(rv_v, spm_v.at[pl.ds(base, _EPT)])
            plsc.subcore_barrier()

            z16f = jnp.zeros((16,), jnp.float32)
            for p in range(2):
                own_lo = (2 * w + p) * _OWN

                def zero(i, carry):
                    for u in range(8):
                        own_v[pl.ds(i * 128 + u * 16, 16)] = z16f
                    return carry
                lax.fori_loop(0, _OWN // 128, zero, 0)

                for h in range(2):
                    pltpu.sync_copy(
                        spm_f.at[pl.ds(h * (_E // 2), _E // 2)], cf_v)
                    pltpu.sync_copy(
                        spm_v.at[pl.ds(h * (_E // 2), _E // 2)], cv_v)

                    def scan(g8, carry):
                        for u in range(8):
                            o = g8 * 128 + u * 16
                            fv = cf_v[pl.ds(o, 16)]
                            vv = cv_v[pl.ds(o, 16)]
                            m = (fv >= own_lo) & (fv < own_lo + _OWN)
                            idx = jnp.where(m, fv - own_lo, 0)
                            plsc.addupdate_scatter(own_v, [idx], vv, mask=m)
                        return carry
                    lax.fori_loop(0, _E // 2 // 128, scan, 0)
                pltpu.sync_copy(
                    own_v, o_hbm.at[pl.ds(l * _N * _N + own_lo, _OWN)])

    return k(s1, d1, s2, d2)


def _mm_nn(a, b, bias=None, relu=False, bm=512, bn=512,
           out_dtype=jnp.float32):
    M, K = a.shape
    _, Nn = b.shape
    in_specs = [pl.BlockSpec((bm, K), lambda i, j: (i, 0)),
                pl.BlockSpec((K, bn), lambda i, j: (0, j))]
    args = [a, b]
    if bias is not None:
        in_specs.append(pl.BlockSpec((1, bn), lambda i, j: (0, j)))
        args.append(bias.reshape(1, Nn))

    def body(a_ref, b_ref, *rest):
        out_ref = rest[-1]
        acc = jnp.dot(a_ref[...], b_ref[...],
                      preferred_element_type=jnp.float32)
        if bias is not None:
            acc = acc + rest[0][...]
        if relu:
            acc = jnp.maximum(acc, 0.0)
        out_ref[...] = acc.astype(out_dtype)

    return pl.pallas_call(
        body,
        grid=(M // bm, Nn // bn),
        in_specs=in_specs,
        out_specs=pl.BlockSpec((bm, bn), lambda i, j: (i, j)),
        out_shape=jax.ShapeDtypeStruct((M, Nn), out_dtype),
        compiler_params=pltpu.CompilerParams(
            dimension_semantics=("parallel", "parallel")),
    )(*args)


def _gate_batched(proj, Cs, adjT, bm=512, bn=512):
    rb = _N // bm

    def body(xj_ref, xi_ref, c_ref, adj_ref, out_ref):
        acc = lax.dot_general(xj_ref[...], xi_ref[...],
                              (((1,), (1,)), ((), ())),
                              preferred_element_type=jnp.float32)
        sig = jax.nn.sigmoid(acc * _SCALE)
        out_ref[0] = (c_ref[0] * adj_ref[...] * sig).astype(jnp.bfloat16)

    return pl.pallas_call(
        body,
        grid=(2, rb, _N // bn),
        in_specs=[
            pl.BlockSpec((bm, _D), lambda b, i, j: (b * rb + i, 1)),
            pl.BlockSpec((bn, _D), lambda b, i, j: (b * rb + j, 0)),
            pl.BlockSpec((1, bm, bn), lambda b, i, j: (b, i, j)),
            pl.BlockSpec((bm, bn), lambda b, i, j: (i, j)),
        ],
        out_specs=pl.BlockSpec((1, bm, bn), lambda b, i, j: (b, i, j)),
        out_shape=jax.ShapeDtypeStruct((2, _N, _N), jnp.bfloat16),
        compiler_params=pltpu.CompilerParams(
            dimension_semantics=("parallel", "parallel", "parallel")),
    )(proj, proj, Cs, adjT)


def _conv_batched(Bs, proj, bias, bm=512, bn=512):
    cb = _D // bn

    def body(b_ref, h_ref, bias_ref, out_ref):
        acc = jnp.dot(b_ref[0], h_ref[...],
                      preferred_element_type=jnp.float32)
        acc = jnp.maximum(acc + bias_ref[...], 0.0)
        out_ref[0] = acc.astype(jnp.bfloat16)

    return pl.pallas_call(
        body,
        grid=(2, _N // bm, cb),
        in_specs=[
            pl.BlockSpec((1, bm, _N), lambda b, i, j: (b, i, 0)),
            pl.BlockSpec((_N, bn), lambda b, i, j: (b, 2 * cb + j)),
            pl.BlockSpec((1, bn), lambda b, i, j: (0, j)),
        ],
        out_specs=pl.BlockSpec((1, bm, bn), lambda b, i, j: (b, i, j)),
        out_shape=jax.ShapeDtypeStruct((2, _N, _D), jnp.bfloat16),
        compiler_params=pltpu.CompilerParams(
            dimension_semantics=("parallel", "parallel", "parallel")),
    )(Bs, proj, bias.reshape(1, _D))


def _mm_nt(a, b, mode, gates=(), bm=512, bn=512,
           out_dtype=jnp.float32):
    M, K = a.shape
    Nn, _ = b.shape
    in_specs = [pl.BlockSpec((bm, K), lambda i, j: (i, 0)),
                pl.BlockSpec((bn, K), lambda i, j: (j, 0))]
    for _g in gates:
        in_specs.append(pl.BlockSpec((bm, bn), lambda i, j: (i, j)))
    if mode == "dual":
        out_shape = (jax.ShapeDtypeStruct((M, Nn), jnp.float32),
                     jax.ShapeDtypeStruct((M, Nn), out_dtype))
        out_specs = (pl.BlockSpec((bm, bn), lambda i, j: (i, j)),
                     pl.BlockSpec((bm, bn), lambda i, j: (i, j)))
    else:
        out_shape = jax.ShapeDtypeStruct((M, Nn), out_dtype)
        out_specs = pl.BlockSpec((bm, bn), lambda i, j: (i, j))

    ng = len(gates)

    def body(a_ref, b_ref, *rest):
        acc = lax.dot_general(a_ref[...], b_ref[...],
                              (((1,), (1,)), ((), ())),
                              preferred_element_type=jnp.float32)
        if mode == "scale":
            rest[-1][...] = (acc * _SCALE).astype(out_dtype)
        elif mode == "gate":
            sig = jax.nn.sigmoid(acc * _SCALE)
            rest[-1][...] = (rest[0][...] * rest[1][...] * sig
                             ).astype(out_dtype)
        else:
            sig = jax.nn.sigmoid(acc * _SCALE)
            rest[ng][...] = sig
            rest[ng + 1][...] = (rest[0][...] * sig).astype(out_dtype)

    return pl.pallas_call(
        body,
        grid=(M // bm, Nn // bn),
        in_specs=in_specs,
        out_specs=out_specs,
        out_shape=out_shape,
        compiler_params=pltpu.CompilerParams(
            dimension_semantics=("parallel", "parallel")),
    )(a, b, *gates)


def _softmax_mm_t(S, V, addend, br=256):
    K, M = S.shape
    _, Dv = V.shape

    def body(s_ref, v_ref, add_ref, out_ref):
        s = s_ref[...].astype(jnp.float32)
        m = jnp.max(s, axis=0, keepdims=True)
        e = jnp.exp(s - m)
        den = jnp.sum(e, axis=0, keepdims=True)
        p = (e / den).astype(v_ref.dtype)
        acc = add_ref[...].astype(jnp.float32) + lax.dot_general(
            p, v_ref[...], (((0,), (0,)), ((), ())),
            preferred_element_type=jnp.float32)
        out_ref[...] = acc.astype(out_ref.dtype)

    return pl.pallas_call(
        body,
        grid=(M // br,),
        in_specs=[pl.BlockSpec((K, br), lambda i: (0, i)),
                  pl.BlockSpec((K, Dv), lambda i: (0, 0)),
                  pl.BlockSpec((br, Dv), lambda i: (i, 0))],
        out_specs=pl.BlockSpec((br, Dv), lambda i: (i, 0)),
        out_shape=jax.ShapeDtypeStruct((M, Dv), jnp.bfloat16),
        compiler_params=pltpu.CompilerParams(
            dimension_semantics=("arbitrary",)),
    )(S, V, addend)


def _softmax_mm(S, V, addend, br=256):
    M, K = S.shape
    _, Dv = V.shape

    def body(s_ref, v_ref, add_ref, out_ref):
        s = s_ref[...].astype(jnp.float32)
        m = jnp.max(s, axis=1, keepdims=True)
        e = jnp.exp(s - m)
        den = jnp.sum(e, axis=1, keepdims=True)
        p = (e / den).astype(v_ref.dtype)
        acc = add_ref[...].astype(jnp.float32) + jnp.dot(
            p, v_ref[...], preferred_element_type=jnp.float32)
        out_ref[...] = acc.astype(out_ref.dtype)

    return pl.pallas_call(
        body,
        grid=(M // br,),
        in_specs=[pl.BlockSpec((br, K), lambda i: (i, 0)),
                  pl.BlockSpec((K, Dv), lambda i: (0, 0)),
                  pl.BlockSpec((br, Dv), lambda i: (i, 0))],
        out_specs=pl.BlockSpec((br, Dv), lambda i: (i, 0)),
        out_shape=jax.ShapeDtypeStruct((M, Dv), jnp.bfloat16),
        compiler_params=pltpu.CompilerParams(
            dimension_semantics=("arbitrary",)),
    )(S, V, addend)


def kernel(x_g1, y_g2, edge_index_g1, edge_index_g2, base_adj,
           Wi, Wj, W1, b1, W2, b2, Wc):
    s1, d1 = edge_index_g1[0], edge_index_g1[1]
    s2, d2 = edge_index_g2[0], edge_index_g2[1]
    counts = _sc_counts(s1, d1, s2, d2)
    bf = jnp.bfloat16
    Cs = counts.reshape(2, _N, _N).astype(bf)
    C1t, C2t = Cs[0], Cs[1]

    xy = jnp.concatenate([x_g1, y_g2], axis=0).astype(bf)
    Wcat = jnp.concatenate([Wi, Wj, W1], axis=1).astype(bf)
    proj = _mm_nn(xy, Wcat, bm=1024, bn=1024, out_dtype=bf)

    adjT = base_adj.T.astype(bf)
    Bs = _gate_batched(proj, Cs, adjT)
    cs = _conv_batched(Bs, proj, b1)
    c1, c2 = cs[0], cs[1]

    t = _mm_nn(c1, Wc.astype(bf), bm=1024, bn=1024, out_dtype=bf)
    S = _mm_nt(t, c2, "scale", out_dtype=bf)
    xn = _softmax_mm(S, c2, c1)
    yn = _softmax_mm_t(S, c1, c2)

    attr1, B1p = _mm_nt(xn, xn, "dual", gates=(C1t,), out_dtype=bf)
    attr2, B2p = _mm_nt(yn, yn, "dual", gates=(C2t,), out_dtype=bf)
    W2b = W2.astype(bf)
    hx = _mm_nn(xn, W2b, bm=1024, bn=1024, out_dtype=bf)
    hy = _mm_nn(yn, W2b, bm=1024, bn=1024, out_dtype=bf)
    o1 = _mm_nn(B1p, hx, bias=b2)
    o2 = _mm_nn(B2p, hy, bias=b2)
    return (o1, o2, edge_index_g1, edge_index_g2, attr1, attr2)


# speedup vs baseline: 1.2149x; 1.2149x over previous
import functools

import jax
import jax.numpy as jnp
from jax import lax
from jax.experimental import pallas as pl
from jax.experimental.pallas import tpu as pltpu
from jax.experimental.pallas import tpu_sc as plsc

_N = 2048
_E = 32768
_D = 1024
_SCALE = 1.0 / float(_D) ** 0.5


_EPT = _E // 16
_NGRP = _EPT // 16
_OWN = 65536


def _sc_counts(s1, d1, s2, d2):
    mesh = plsc.VectorSubcoreMesh(core_axis_name="c", subcore_axis_name="s")
    out_t = (jax.ShapeDtypeStruct((_N * _N,), jnp.float32),
             jax.ShapeDtypeStruct((_N * _N,), jnp.float32))
    scratch = [
        pltpu.VMEM((_EPT,), jnp.int32),
        pltpu.VMEM((_EPT,), jnp.int32),
        pltpu.VMEM((_EPT,), jnp.int32),
        pltpu.VMEM((_EPT,), jnp.float32),
        pltpu.VMEM((_E // 2,), jnp.int32),
        pltpu.VMEM((_E // 2,), jnp.float32),
        pltpu.VMEM((_OWN,), jnp.float32),
        pltpu.VMEM((32,), jnp.int32),
        pltpu.VMEM_SHARED((_E,), jnp.int32),
        pltpu.VMEM_SHARED((_E,), jnp.float32),
    ]
    @functools.partial(pl.kernel, out_type=out_t, mesh=mesh,
                       scratch_types=scratch,
                       compiler_params=pltpu.CompilerParams(
                           needs_layout_passes=False))
    def k(s1_hbm, d1_hbm, s2_hbm, d2_hbm, o1_hbm, o2_hbm,
          src_v, dst_v, fs_v, rv_v, cf_v, cv_v, own_v, rot_v, spm_f, spm_v):
        c = lax.axis_index("c")
        t = lax.axis_index("s")
        w = c * 16 + t
        base = t * _EPT
        iota16 = lax.iota(jnp.int32, 16)

        for l in range(2):
            sh = (s1_hbm, s2_hbm)[l]
            dh = (d1_hbm, d2_hbm)[l]
            out_hbm = (o1_hbm, o2_hbm)[l]
            pltpu.sync_copy(sh.at[pl.ds(base, _EPT)], src_v)
            pltpu.sync_copy(dh.at[pl.ds(base, _EPT)], dst_v)

            def pre(g, carry):
                sv = src_v[pl.ds(g * 16, 16)]
                dv = dst_v[pl.ds(g * 16, 16)]
                f = dv * _N + sv
                rot_v[pl.ds(0, 16)] = f
                rot_v[pl.ds(16, 16)] = f
                eqs = [f == rot_v[pl.ds(kk, 16)] for kk in range(1, 16)]
                ints = [jnp.where(e, 1, 0) for e in eqs]
                dups = [eqs[kk - 1] & (iota16 >= 16 - kk)
                        for kk in range(1, 16)]
                while len(ints) > 1:
                    ints = [a + b for a, b in zip(ints[::2], ints[1::2])
                            ] + ints[len(ints) & ~1:]
                    dups = [a | b for a, b in zip(dups[::2], dups[1::2])
                            ] + dups[len(dups) & ~1:]
                cnt = 1 + ints[0]
                dup = dups[0]
                fs_v[pl.ds(g * 16, 16)] = jnp.where(dup, _N * _N, f)
                rv_v[pl.ds(g * 16, 16)] = cnt.astype(jnp.float32)
                return carry
            lax.fori_loop(0, _NGRP, pre, 0)

            plsc.subcore_barrier()
            pltpu.sync_copy(fs_v, spm_f.at[pl.ds(base, _EPT)])
            pltpu.sync_copy(rv_v, spm_v.at[pl.ds(base, _EPT)])
            plsc.subcore_barrier()

            z16f = jnp.zeros((16,), jnp.float32)
            for p in range(2):
                own_lo = (2 * w + p) * _OWN

                def zero(i, carry):
                    for u in range(8):
                        own_v[pl.ds(i * 128 + u * 16, 16)] = z16f
                    return carry
                lax.fori_loop(0, _OWN // 128, zero, 0)

                for h in range(2):
                    pltpu.sync_copy(
                        spm_f.at[pl.ds(h * (_E // 2), _E // 2)], cf_v)
                    pltpu.sync_copy(
                        spm_v.at[pl.ds(h * (_E // 2), _E // 2)], cv_v)

                    def scan(g8, carry):
                        for u in range(8):
                            o = g8 * 128 + u * 16
                            fv = cf_v[pl.ds(o, 16)]
                            vv = cv_v[pl.ds(o, 16)]
                            m = (fv >= own_lo) & (fv < own_lo + _OWN)
                            idx = jnp.where(m, fv - own_lo, 0)
                            plsc.addupdate_scatter(own_v, [idx], vv, mask=m)
                        return carry
                    lax.fori_loop(0, _E // 2 // 128, scan, 0)
                pltpu.sync_copy(own_v, out_hbm.at[pl.ds(own_lo, _OWN)])

    return k(s1, d1, s2, d2)


def _mm_nn(a, b, bias=None, relu=False, bm=512, bn=512,
           out_dtype=jnp.float32):
    M, K = a.shape
    _, Nn = b.shape
    in_specs = [pl.BlockSpec((bm, K), lambda i, j: (i, 0)),
                pl.BlockSpec((K, bn), lambda i, j: (0, j))]
    args = [a, b]
    if bias is not None:
        in_specs.append(pl.BlockSpec((1, bn), lambda i, j: (0, j)))
        args.append(bias.reshape(1, Nn))

    def body(a_ref, b_ref, *rest):
        out_ref = rest[-1]
        acc = jnp.dot(a_ref[...], b_ref[...],
                      preferred_element_type=jnp.float32)
        if bias is not None:
            acc = acc + rest[0][...]
        if relu:
            acc = jnp.maximum(acc, 0.0)
        out_ref[...] = acc.astype(out_dtype)

    return pl.pallas_call(
        body,
        grid=(M // bm, Nn // bn),
        in_specs=in_specs,
        out_specs=pl.BlockSpec((bm, bn), lambda i, j: (i, j)),
        out_shape=jax.ShapeDtypeStruct((M, Nn), out_dtype),
        compiler_params=pltpu.CompilerParams(
            dimension_semantics=("parallel", "parallel")),
    )(*args)


def _gate_batched(proj, Cs, adjT, bm=512, bn=512):
    rb = _N // bm

    def body(xj_ref, xi_ref, c_ref, adj_ref, out_ref):
        acc = lax.dot_general(xj_ref[...], xi_ref[...],
                              (((1,), (1,)), ((), ())),
                              preferred_element_type=jnp.float32)
        sig = jax.nn.sigmoid(acc * _SCALE)
        out_ref[0] = (c_ref[0] * adj_ref[...] * sig).astype(jnp.bfloat16)

    return pl.pallas_call(
        body,
        grid=(2, rb, _N // bn),
        in_specs=[
            pl.BlockSpec((bm, _D), lambda b, i, j: (b * rb + i, 1)),
            pl.BlockSpec((bn, _D), lambda b, i, j: (b * rb + j, 0)),
            pl.BlockSpec((1, bm, bn), lambda b, i, j: (b, i, j)),
            pl.BlockSpec((bm, bn), lambda b, i, j: (i, j)),
        ],
        out_specs=pl.BlockSpec((1, bm, bn), lambda b, i, j: (b, i, j)),
        out_shape=jax.ShapeDtypeStruct((2, _N, _N), jnp.bfloat16),
        compiler_params=pltpu.CompilerParams(
            dimension_semantics=("parallel", "parallel", "parallel")),
    )(proj, proj, Cs, adjT)


def _conv_batched(Bs, proj, bias, bm=512, bn=512):
    cb = _D // bn

    def body(b_ref, h_ref, bias_ref, out_ref):
        acc = jnp.dot(b_ref[0], h_ref[...],
                      preferred_element_type=jnp.float32)
        acc = jnp.maximum(acc + bias_ref[...], 0.0)
        out_ref[0] = acc.astype(jnp.bfloat16)

    return pl.pallas_call(
        body,
        grid=(2, _N // bm, cb),
        in_specs=[
            pl.BlockSpec((1, bm, _N), lambda b, i, j: (b, i, 0)),
            pl.BlockSpec((_N, bn), lambda b, i, j: (b, 2 * cb + j)),
            pl.BlockSpec((1, bn), lambda b, i, j: (0, j)),
        ],
        out_specs=pl.BlockSpec((1, bm, bn), lambda b, i, j: (b, i, j)),
        out_shape=jax.ShapeDtypeStruct((2, _N, _D), jnp.bfloat16),
        compiler_params=pltpu.CompilerParams(
            dimension_semantics=("parallel", "parallel", "parallel")),
    )(Bs, proj, bias.reshape(1, _D))


def _mm_nt(a, b, mode, gates=(), bm=512, bn=512,
           out_dtype=jnp.float32):
    M, K = a.shape
    Nn, _ = b.shape
    in_specs = [pl.BlockSpec((bm, K), lambda i, j: (i, 0)),
                pl.BlockSpec((bn, K), lambda i, j: (j, 0))]
    for _g in gates:
        in_specs.append(pl.BlockSpec((bm, bn), lambda i, j: (i, j)))
    if mode == "dual":
        out_shape = (jax.ShapeDtypeStruct((M, Nn), jnp.float32),
                     jax.ShapeDtypeStruct((M, Nn), out_dtype))
        out_specs = (pl.BlockSpec((bm, bn), lambda i, j: (i, j)),
                     pl.BlockSpec((bm, bn), lambda i, j: (i, j)))
    else:
        out_shape = jax.ShapeDtypeStruct((M, Nn), out_dtype)
        out_specs = pl.BlockSpec((bm, bn), lambda i, j: (i, j))

    ng = len(gates)

    def body(a_ref, b_ref, *rest):
        acc = lax.dot_general(a_ref[...], b_ref[...],
                              (((1,), (1,)), ((), ())),
                              preferred_element_type=jnp.float32)
        if mode == "scale":
            rest[-1][...] = (acc * _SCALE).astype(out_dtype)
        elif mode == "gate":
            sig = jax.nn.sigmoid(acc * _SCALE)
            rest[-1][...] = (rest[0][...] * rest[1][...] * sig
                             ).astype(out_dtype)
        else:
            sig = jax.nn.sigmoid(acc * _SCALE)
            rest[ng][...] = sig
            rest[ng + 1][...] = (rest[0][...] * sig).astype(out_dtype)

    return pl.pallas_call(
        body,
        grid=(M // bm, Nn // bn),
        in_specs=in_specs,
        out_specs=out_specs,
        out_shape=out_shape,
        compiler_params=pltpu.CompilerParams(
            dimension_semantics=("parallel", "parallel")),
    )(a, b, *gates)


def _softmax_mm_t(S, V, addend, br=256):
    K, M = S.shape
    _, Dv = V.shape

    def body(s_ref, v_ref, add_ref, out_ref):
        s = s_ref[...].astype(jnp.float32)
        m = jnp.max(s, axis=0, keepdims=True)
        e = jnp.exp(s - m)
        den = jnp.sum(e, axis=0, keepdims=True)
        p = (e / den).astype(v_ref.dtype)
        acc = add_ref[...].astype(jnp.float32) + lax.dot_general(
            p, v_ref[...], (((0,), (0,)), ((), ())),
            preferred_element_type=jnp.float32)
        out_ref[...] = acc.astype(out_ref.dtype)

    return pl.pallas_call(
        body,
        grid=(M // br,),
        in_specs=[pl.BlockSpec((K, br), lambda i: (0, i)),
                  pl.BlockSpec((K, Dv), lambda i: (0, 0)),
                  pl.BlockSpec((br, Dv), lambda i: (i, 0))],
        out_specs=pl.BlockSpec((br, Dv), lambda i: (i, 0)),
        out_shape=jax.ShapeDtypeStruct((M, Dv), jnp.bfloat16),
        compiler_params=pltpu.CompilerParams(
            dimension_semantics=("arbitrary",)),
    )(S, V, addend)


def _softmax_mm(S, V, addend, br=256):
    M, K = S.shape
    _, Dv = V.shape

    def body(s_ref, v_ref, add_ref, out_ref):
        s = s_ref[...].astype(jnp.float32)
        m = jnp.max(s, axis=1, keepdims=True)
        e = jnp.exp(s - m)
        den = jnp.sum(e, axis=1, keepdims=True)
        p = (e / den).astype(v_ref.dtype)
        acc = add_ref[...].astype(jnp.float32) + jnp.dot(
            p, v_ref[...], preferred_element_type=jnp.float32)
        out_ref[...] = acc.astype(out_ref.dtype)

    return pl.pallas_call(
        body,
        grid=(M // br,),
        in_specs=[pl.BlockSpec((br, K), lambda i: (i, 0)),
                  pl.BlockSpec((K, Dv), lambda i: (0, 0)),
                  pl.BlockSpec((br, Dv), lambda i: (i, 0))],
        out_specs=pl.BlockSpec((br, Dv), lambda i: (i, 0)),
        out_shape=jax.ShapeDtypeStruct((M, Dv), jnp.bfloat16),
        compiler_params=pltpu.CompilerParams(
            dimension_semantics=("arbitrary",)),
    )(S, V, addend)


def kernel(x_g1, y_g2, edge_index_g1, edge_index_g2, base_adj,
           Wi, Wj, W1, b1, W2, b2, Wc):
    s1, d1 = edge_index_g1[0], edge_index_g1[1]
    s2, d2 = edge_index_g2[0], edge_index_g2[1]
    cnt1, cnt2 = _sc_counts(s1, d1, s2, d2)
    bf = jnp.bfloat16
    C1t = cnt1.reshape(_N, _N).astype(bf)
    C2t = cnt2.reshape(_N, _N).astype(bf)

    xy = jnp.concatenate([x_g1, y_g2], axis=0).astype(bf)
    Wcat = jnp.concatenate([Wi, Wj, W1], axis=1).astype(bf)
    proj = _mm_nn(xy, Wcat, bm=1024, bn=1024, out_dtype=bf)
    xi1, xj1, h1 = proj[:_N, :_D], proj[:_N, _D:2 * _D], proj[:_N, 2 * _D:]
    xi2, xj2, h2 = proj[_N:, :_D], proj[_N:, _D:2 * _D], proj[_N:, 2 * _D:]

    adjT = base_adj.T.astype(bf)
    B1 = _mm_nt(xj1, xi1, "gate", gates=(C1t, adjT), bm=1024, bn=1024,
                out_dtype=bf)
    B2 = _mm_nt(xj2, xi2, "gate", gates=(C2t, adjT), bm=1024, bn=1024,
                out_dtype=bf)
    c1 = _mm_nn(B1, h1, bias=b1, relu=True, bm=1024, bn=1024, out_dtype=bf)
    c2 = _mm_nn(B2, h2, bias=b1, relu=True, bm=1024, bn=1024, out_dtype=bf)

    t = _mm_nn(c1, Wc.astype(bf), bm=1024, bn=1024, out_dtype=bf)
    S = _mm_nt(t, c2, "scale", bm=1024, bn=1024, out_dtype=bf)
    xn = _softmax_mm(S, c2, c1, br=512)
    yn = _softmax_mm_t(S, c1, c2, br=512)

    attr1, B1p = _mm_nt(xn, xn, "dual", gates=(C1t,), bm=1024, bn=1024,
                        out_dtype=bf)
    attr2, B2p = _mm_nt(yn, yn, "dual", gates=(C2t,), bm=1024, bn=1024,
                        out_dtype=bf)
    W2b = W2.astype(bf)
    hx = _mm_nn(xn, W2b, bm=1024, bn=1024, out_dtype=bf)
    hy = _mm_nn(yn, W2b, bm=1024, bn=1024, out_dtype=bf)
    o1 = _mm_nn(B1p, hx, bias=b2, bm=1024, bn=1024)
    o2 = _mm_nn(B2p, hy, bias=b2, bm=1024, bn=1024)
    return (o1, o2, edge_index_g1, edge_index_g2, attr1, attr2)


# speedup vs baseline: 1.2288x; 1.0114x over previous
import functools

import jax
import jax.numpy as jnp
from jax import lax
from jax.experimental import pallas as pl
from jax.experimental.pallas import tpu as pltpu
from jax.experimental.pallas import tpu_sc as plsc

_N = 2048
_E = 32768
_D = 1024
_SCALE = 1.0 / float(_D) ** 0.5


_EPT = _E // 16
_NGRP = _EPT // 16
_OWN = 65536


def _sc_counts(s1, d1, s2, d2):
    mesh = plsc.VectorSubcoreMesh(core_axis_name="c", subcore_axis_name="s")
    out_t = (jax.ShapeDtypeStruct((_N * _N,), jnp.float32),
             jax.ShapeDtypeStruct((_N * _N,), jnp.float32))
    scratch = [
        pltpu.VMEM((_EPT,), jnp.int32),
        pltpu.VMEM((_EPT,), jnp.int32),
        pltpu.VMEM((_EPT,), jnp.int32),
        pltpu.VMEM((_EPT,), jnp.float32),
        pltpu.VMEM((_E // 4,), jnp.int32),
        pltpu.VMEM((_E // 4,), jnp.float32),
        pltpu.VMEM((_E // 4,), jnp.int32),
        pltpu.VMEM((_E // 4,), jnp.float32),
        pltpu.VMEM((_OWN,), jnp.float32),
        pltpu.VMEM((32,), jnp.int32),
        pltpu.VMEM_SHARED((_E,), jnp.int32),
        pltpu.VMEM_SHARED((_E,), jnp.float32),
        pltpu.SemaphoreType.DMA,
        pltpu.SemaphoreType.DMA,
        pltpu.SemaphoreType.DMA,
        pltpu.SemaphoreType.DMA,
    ]
    @functools.partial(pl.kernel, out_type=out_t, mesh=mesh,
                       scratch_types=scratch,
                       compiler_params=pltpu.CompilerParams(
                           needs_layout_passes=False))
    def k(s1_hbm, d1_hbm, s2_hbm, d2_hbm, o1_hbm, o2_hbm,
          src_v, dst_v, fs_v, rv_v, cfa_v, cva_v, cfb_v, cvb_v, own_v,
          rot_v, spm_f, spm_v, sfa, sva, sfb, svb):
        c = lax.axis_index("c")
        t = lax.axis_index("s")
        w = c * 16 + t
        base = t * _EPT
        iota16 = lax.iota(jnp.int32, 16)

        for l in range(2):
            sh = (s1_hbm, s2_hbm)[l]
            dh = (d1_hbm, d2_hbm)[l]
            out_hbm = (o1_hbm, o2_hbm)[l]
            pltpu.sync_copy(sh.at[pl.ds(base, _EPT)], src_v)
            pltpu.sync_copy(dh.at[pl.ds(base, _EPT)], dst_v)

            def pre(g, carry):
                sv = src_v[pl.ds(g * 16, 16)]
                dv = dst_v[pl.ds(g * 16, 16)]
                f = dv * _N + sv
                rot_v[pl.ds(0, 16)] = f
                rot_v[pl.ds(16, 16)] = f
                eqs = [f == rot_v[pl.ds(kk, 16)] for kk in range(1, 16)]
                ints = [jnp.where(e, 1, 0) for e in eqs]
                dups = [eqs[kk - 1] & (iota16 >= 16 - kk)
                        for kk in range(1, 16)]
                while len(ints) > 1:
                    ints = [a + b for a, b in zip(ints[::2], ints[1::2])
                            ] + ints[len(ints) & ~1:]
                    dups = [a | b for a, b in zip(dups[::2], dups[1::2])
                            ] + dups[len(dups) & ~1:]
                cnt = 1 + ints[0]
                dup = dups[0]
                fs_v[pl.ds(g * 16, 16)] = jnp.where(dup, _N * _N, f)
                rv_v[pl.ds(g * 16, 16)] = cnt.astype(jnp.float32)
                return carry
            lax.fori_loop(0, _NGRP, pre, 0)

            plsc.subcore_barrier()
            pltpu.sync_copy(fs_v, spm_f.at[pl.ds(base, _EPT)])
            pltpu.sync_copy(rv_v, spm_v.at[pl.ds(base, _EPT)])
            plsc.subcore_barrier()

            z16f = jnp.zeros((16,), jnp.float32)
            for p in range(2):
                own_lo = (2 * w + p) * _OWN

                def zero(i, carry):
                    for u in range(8):
                        own_v[pl.ds(i * 128 + u * 16, 16)] = z16f
                    return carry
                lax.fori_loop(0, _OWN // 128, zero, 0)

                qs = _E // 4
                bufs = ((cfa_v, cva_v, sfa, sva), (cfb_v, cvb_v, sfb, svb))

                def start(q, bset):
                    cf, cv, sf, sv = bset
                    return (pltpu.async_copy(
                                spm_f.at[pl.ds(q * qs, qs)], cf, sf),
                            pltpu.async_copy(
                                spm_v.at[pl.ds(q * qs, qs)], cv, sv))
                pend = start(0, bufs[0])
                for q in range(4):
                    cf_v, cv_v = bufs[q % 2][:2]
                    pend[0].wait()
                    pend[1].wait()
                    if q < 3:
                        pend = start(q + 1, bufs[(q + 1) % 2])

                    def scan(g8, carry, cf_v=cf_v, cv_v=cv_v):
                        for u in range(8):
                            o = g8 * 128 + u * 16
                            fv = cf_v[pl.ds(o, 16)]
                            vv = cv_v[pl.ds(o, 16)]
                            m = (fv >= own_lo) & (fv < own_lo + _OWN)
                            idx = jnp.where(m, fv - own_lo, 0)
                            plsc.addupdate_scatter(own_v, [idx], vv, mask=m)
                        return carry
                    lax.fori_loop(0, qs // 128, scan, 0)
                pltpu.sync_copy(own_v, out_hbm.at[pl.ds(own_lo, _OWN)])

    return k(s1, d1, s2, d2)


def _mm_nn(a, b, bias=None, relu=False, bm=512, bn=512,
           out_dtype=jnp.float32):
    M, K = a.shape
    _, Nn = b.shape
    in_specs = [pl.BlockSpec((bm, K), lambda i, j: (i, 0)),
                pl.BlockSpec((K, bn), lambda i, j: (0, j))]
    args = [a, b]
    if bias is not None:
        in_specs.append(pl.BlockSpec((1, bn), lambda i, j: (0, j)))
        args.append(bias.reshape(1, Nn))

    def body(a_ref, b_ref, *rest):
        out_ref = rest[-1]
        acc = jnp.dot(a_ref[...], b_ref[...],
                      preferred_element_type=jnp.float32)
        if bias is not None:
            acc = acc + rest[0][...]
        if relu:
            acc = jnp.maximum(acc, 0.0)
        out_ref[...] = acc.astype(out_dtype)

    return pl.pallas_call(
        body,
        grid=(M // bm, Nn // bn),
        in_specs=in_specs,
        out_specs=pl.BlockSpec((bm, bn), lambda i, j: (i, j)),
        out_shape=jax.ShapeDtypeStruct((M, Nn), out_dtype),
        compiler_params=pltpu.CompilerParams(
            dimension_semantics=("parallel", "parallel")),
    )(*args)


def _gate_batched(proj, Cs, adjT, bm=512, bn=512):
    rb = _N // bm

    def body(xj_ref, xi_ref, c_ref, adj_ref, out_ref):
        acc = lax.dot_general(xj_ref[...], xi_ref[...],
                              (((1,), (1,)), ((), ())),
                              preferred_element_type=jnp.float32)
        sig = jax.nn.sigmoid(acc * _SCALE)
        out_ref[0] = (c_ref[0] * adj_ref[...] * sig).astype(jnp.bfloat16)

    return pl.pallas_call(
        body,
        grid=(2, rb, _N // bn),
        in_specs=[
            pl.BlockSpec((bm, _D), lambda b, i, j: (b * rb + i, 1)),
            pl.BlockSpec((bn, _D), lambda b, i, j: (b * rb + j, 0)),
            pl.BlockSpec((1, bm, bn), lambda b, i, j: (b, i, j)),
            pl.BlockSpec((bm, bn), lambda b, i, j: (i, j)),
        ],
        out_specs=pl.BlockSpec((1, bm, bn), lambda b, i, j: (b, i, j)),
        out_shape=jax.ShapeDtypeStruct((2, _N, _N), jnp.bfloat16),
        compiler_params=pltpu.CompilerParams(
            dimension_semantics=("parallel", "parallel", "parallel")),
    )(proj, proj, Cs, adjT)


def _conv_batched(Bs, proj, bias, bm=512, bn=512):
    cb = _D // bn

    def body(b_ref, h_ref, bias_ref, out_ref):
        acc = jnp.dot(b_ref[0], h_ref[...],
                      preferred_element_type=jnp.float32)
        acc = jnp.maximum(acc + bias_ref[...], 0.0)
        out_ref[0] = acc.astype(jnp.bfloat16)

    return pl.pallas_call(
        body,
        grid=(2, _N // bm, cb),
        in_specs=[
            pl.BlockSpec((1, bm, _N), lambda b, i, j: (b, i, 0)),
            pl.BlockSpec((_N, bn), lambda b, i, j: (b, 2 * cb + j)),
            pl.BlockSpec((1, bn), lambda b, i, j: (0, j)),
        ],
        out_specs=pl.BlockSpec((1, bm, bn), lambda b, i, j: (b, i, j)),
        out_shape=jax.ShapeDtypeStruct((2, _N, _D), jnp.bfloat16),
        compiler_params=pltpu.CompilerParams(
            dimension_semantics=("parallel", "parallel", "parallel")),
    )(Bs, proj, bias.reshape(1, _D))


def _mm_nt(a, b, mode, gates=(), bm=512, bn=512,
           out_dtype=jnp.float32):
    M, K = a.shape
    Nn, _ = b.shape
    in_specs = [pl.BlockSpec((bm, K), lambda i, j: (i, 0)),
                pl.BlockSpec((bn, K), lambda i, j: (j, 0))]
    for _g in gates:
        in_specs.append(pl.BlockSpec((bm, bn), lambda i, j: (i, j)))
    if mode == "dual":
        out_shape = (jax.ShapeDtypeStruct((M, Nn), jnp.float32),
                     jax.ShapeDtypeStruct((M, Nn), out_dtype))
        out_specs = (pl.BlockSpec((bm, bn), lambda i, j: (i, j)),
                     pl.BlockSpec((bm, bn), lambda i, j: (i, j)))
    else:
        out_shape = jax.ShapeDtypeStruct((M, Nn), out_dtype)
        out_specs = pl.BlockSpec((bm, bn), lambda i, j: (i, j))

    ng = len(gates)

    def body(a_ref, b_ref, *rest):
        acc = lax.dot_general(a_ref[...], b_ref[...],
                              (((1,), (1,)), ((), ())),
                              preferred_element_type=jnp.float32)
        if mode == "scale":
            rest[-1][...] = (acc * _SCALE).astype(out_dtype)
        elif mode == "gate":
            sig = jax.nn.sigmoid(acc * _SCALE)
            rest[-1][...] = (rest[0][...] * rest[1][...] * sig
                             ).astype(out_dtype)
        else:
            sig = jax.nn.sigmoid(acc * _SCALE)
            rest[ng][...] = sig
            rest[ng + 1][...] = (rest[0][...] * sig).astype(out_dtype)

    return pl.pallas_call(
        body,
        grid=(M // bm, Nn // bn),
        in_specs=in_specs,
        out_specs=out_specs,
        out_shape=out_shape,
        compiler_params=pltpu.CompilerParams(
            dimension_semantics=("parallel", "parallel")),
    )(a, b, *gates)


def _softmax_mm_t(S, V, addend, br=256):
    K, M = S.shape
    _, Dv = V.shape

    def body(s_ref, v_ref, add_ref, out_ref):
        s = s_ref[...].astype(jnp.float32)
        m = jnp.max(s, axis=0, keepdims=True)
        e = jnp.exp(s - m)
        den = jnp.sum(e, axis=0, keepdims=True)
        p = (e / den).astype(v_ref.dtype)
        acc = add_ref[...].astype(jnp.float32) + lax.dot_general(
            p, v_ref[...], (((0,), (0,)), ((), ())),
            preferred_element_type=jnp.float32)
        out_ref[...] = acc.astype(out_ref.dtype)

    return pl.pallas_call(
        body,
        grid=(M // br,),
        in_specs=[pl.BlockSpec((K, br), lambda i: (0, i)),
                  pl.BlockSpec((K, Dv), lambda i: (0, 0)),
                  pl.BlockSpec((br, Dv), lambda i: (i, 0))],
        out_specs=pl.BlockSpec((br, Dv), lambda i: (i, 0)),
        out_shape=jax.ShapeDtypeStruct((M, Dv), jnp.bfloat16),
        compiler_params=pltpu.CompilerParams(
            dimension_semantics=("arbitrary",)),
    )(S, V, addend)


def _softmax_mm(S, V, addend, br=256):
    M, K = S.shape
    _, Dv = V.shape

    def body(s_ref, v_ref, add_ref, out_ref):
        s = s_ref[...].astype(jnp.float32)
        m = jnp.max(s, axis=1, keepdims=True)
        e = jnp.exp(s - m)
        den = jnp.sum(e, axis=1, keepdims=True)
        p = (e / den).astype(v_ref.dtype)
        acc = add_ref[...].astype(jnp.float32) + jnp.dot(
            p, v_ref[...], preferred_element_type=jnp.float32)
        out_ref[...] = acc.astype(out_ref.dtype)

    return pl.pallas_call(
        body,
        grid=(M // br,),
        in_specs=[pl.BlockSpec((br, K), lambda i: (i, 0)),
                  pl.BlockSpec((K, Dv), lambda i: (0, 0)),
                  pl.BlockSpec((br, Dv), lambda i: (i, 0))],
        out_specs=pl.BlockSpec((br, Dv), lambda i: (i, 0)),
        out_shape=jax.ShapeDtypeStruct((M, Dv), jnp.bfloat16),
        compiler_params=pltpu.CompilerParams(
            dimension_semantics=("arbitrary",)),
    )(S, V, addend)


def kernel(x_g1, y_g2, edge_index_g1, edge_index_g2, base_adj,
           Wi, Wj, W1, b1, W2, b2, Wc):
    s1, d1 = edge_index_g1[0], edge_index_g1[1]
    s2, d2 = edge_index_g2[0], edge_index_g2[1]
    cnt1, cnt2 = _sc_counts(s1, d1, s2, d2)
    bf = jnp.bfloat16
    C1t = cnt1.reshape(_N, _N).astype(bf)
    C2t = cnt2.reshape(_N, _N).astype(bf)

    xy = jnp.concatenate([x_g1, y_g2], axis=0).astype(bf)
    Wcat = jnp.concatenate([Wi, Wj, W1], axis=1).astype(bf)
    proj = _mm_nn(xy, Wcat, bm=1024, bn=1024, out_dtype=bf)
    xi1, xj1, h1 = proj[:_N, :_D], proj[:_N, _D:2 * _D], proj[:_N, 2 * _D:]
    xi2, xj2, h2 = proj[_N:, :_D], proj[_N:, _D:2 * _D], proj[_N:, 2 * _D:]

    adjT = base_adj.T.astype(bf)
    B1 = _mm_nt(xj1, xi1, "gate", gates=(C1t, adjT), bm=1024, bn=1024,
                out_dtype=bf)
    B2 = _mm_nt(xj2, xi2, "gate", gates=(C2t, adjT), bm=1024, bn=1024,
                out_dtype=bf)
    c1 = _mm_nn(B1, h1, bias=b1, relu=True, bm=1024, bn=1024, out_dtype=bf)
    c2 = _mm_nn(B2, h2, bias=b1, relu=True, bm=1024, bn=1024, out_dtype=bf)

    t = _mm_nn(c1, Wc.astype(bf), bm=1024, bn=1024, out_dtype=bf)
    S = _mm_nt(t, c2, "scale", bm=1024, bn=1024, out_dtype=bf)
    xn = _softmax_mm(S, c2, c1, br=512)
    yn = _softmax_mm_t(S, c1, c2, br=512)

    attr1, B1p = _mm_nt(xn, xn, "dual", gates=(C1t,), bm=1024, bn=1024,
                        out_dtype=bf)
    attr2, B2p = _mm_nt(yn, yn, "dual", gates=(C2t,), bm=1024, bn=1024,
                        out_dtype=bf)
    W2b = W2.astype(bf)
    hx = _mm_nn(xn, W2b, bm=1024, bn=1024, out_dtype=bf)
    hy = _mm_nn(yn, W2b, bm=1024, bn=1024, out_dtype=bf)
    o1 = _mm_nn(B1p, hx, bias=b2, bm=1024, bn=1024)
    o2 = _mm_nn(B2p, hy, bias=b2, bm=1024, bn=1024)
    return (o1, o2, edge_index_g1, edge_index_g2, attr1, attr2)
